# baseline (speedup 1.0000x reference)
import functools

import jax
import jax.numpy as jnp
from jax import lax
from jax.experimental import pallas as pl
from jax.experimental.pallas import tpu as pltpu
from jax.experimental.pallas import tpu_sc as plsc

BATCH = 16384
DIM = 64
ODIM = 2 * DIM

_info = plsc.get_sparse_core_info()
_NC = _info.num_cores
_NS = _info.num_subcores
_NW = _NC * _NS
_BPW = BATCH // _NW
_CH = 128
_NCHUNK = _BPW // _CH
_CPP = 2
_PR = _CPP * _CH
_NPASS = _NCHUNK // _CPP


def _tc_pad(table, rows, blk, left):
    grid = rows // blk
    lo, hi = (DIM, ODIM) if left else (0, DIM)

    def body(in_hbm, out_ref, buf, sem):
        i = pl.program_id(0)
        cp = pltpu.make_async_copy(
            in_hbm.at[pl.ds(i * blk, blk), :], buf, sem)
        cp.start()
        cp.wait()
        out_ref[:, lo:hi] = buf[...]

    return pl.pallas_call(
        body,
        grid=(grid,),
        in_specs=[pl.BlockSpec(memory_space=pl.ANY)],
        out_specs=pl.BlockSpec((blk, ODIM), lambda i: (i, 0)),
        out_shape=jax.ShapeDtypeStruct((rows, ODIM), jnp.float32),
        scratch_shapes=[
            pltpu.VMEM((blk, DIM), jnp.float32),
            pltpu.SemaphoreType.DMA,
        ],
    )(table)


def _build():
    mesh = plsc.VectorSubcoreMesh(core_axis_name="c", subcore_axis_name="s")

    @functools.partial(
        pl.kernel,
        mesh=mesh,
        out_type=jax.ShapeDtypeStruct((BATCH, ODIM), jnp.float32),
        scratch_types=[
            pltpu.VMEM((_NCHUNK, _CH), jnp.int32),
            pltpu.VMEM((_NCHUNK, _CH), jnp.int32),
            pltpu.VMEM((_PR, ODIM), jnp.float32),
            pltpu.VMEM((_PR, ODIM), jnp.float32),
            pltpu.SemaphoreType.DMA,
        ],
    )
    def k(adduct_hbm, instr_hbm, apad_hbm, ipad_hbm, out_hbm,
          aidx_v, iidx_v, a_v, b_v, sem):
        wid = lax.axis_index("s") * _NC + lax.axis_index("c")
        base = wid * _BPW
        row0 = wid * _NCHUNK
        pltpu.sync_copy(adduct_hbm.at[pl.ds(row0, _NCHUNK), :], aidx_v)
        pltpu.sync_copy(instr_hbm.at[pl.ds(row0, _NCHUNK), :], iidx_v)
        for p in range(_NPASS):
            copies = []
            for j in range(_CPP):
                c = p * _CPP + j
                copies.append(pltpu.async_copy(
                    apad_hbm.at[aidx_v.at[c]],
                    a_v.at[pl.ds(j * _CH, _CH)], sem))
                copies.append(pltpu.async_copy(
                    ipad_hbm.at[iidx_v.at[c]],
                    b_v.at[pl.ds(j * _CH, _CH)], sem))
            for cp in copies:
                cp.wait()

            def mergerow(r, _):
                for k16 in range(DIM // 16):
                    sl = pl.ds(DIM + k16 * 16, 16)
                    a_v[r, sl] = b_v[r, sl]
                return ()

            lax.fori_loop(0, _PR, mergerow, ())
            pltpu.sync_copy(a_v, out_hbm.at[pl.ds(base + p * _PR, _PR), :])

    return k


_sc_kernel = _build()


def kernel(adduct, instrument_type, adduct_table, instrument_type_table):
    apad = _tc_pad(adduct_table, rows=100000, blk=5000, left=False)
    ipad = _tc_pad(instrument_type_table, rows=1000, blk=1000, left=True)
    adduct2 = adduct.reshape(_NW * _NCHUNK, _CH)
    instr2 = instrument_type.reshape(_NW * _NCHUNK, _CH)
    return _sc_kernel(adduct2, instr2, apad, ipad)


# speedup vs baseline: 1.4804x; 1.4804x over previous
import functools

import jax
import jax.numpy as jnp
from jax import lax
from jax.experimental import pallas as pl
from jax.experimental.pallas import tpu as pltpu
from jax.experimental.pallas import tpu_sc as plsc

BATCH = 16384
DIM = 64
ODIM = 2 * DIM

_info = plsc.get_sparse_core_info()
_NC = _info.num_cores
_NS = _info.num_subcores
_NW = _NC * _NS
_BPW = BATCH // _NW
_CH = 128
_NCHUNK = _BPW // _CH


def _build():
    mesh = plsc.VectorSubcoreMesh(core_axis_name="c", subcore_axis_name="s")

    @functools.partial(
        pl.kernel,
        mesh=mesh,
        out_type=jax.ShapeDtypeStruct((BATCH, ODIM), jnp.float32),
        scratch_types=[
            pltpu.VMEM((_NCHUNK, _CH), jnp.int32),
            pltpu.VMEM((_NCHUNK, _CH), jnp.int32),
            pltpu.VMEM((_CH, ODIM), jnp.float32),
            pltpu.VMEM((_CH, ODIM), jnp.float32),
            pltpu.VMEM((_CH, ODIM), jnp.float32),
            pltpu.VMEM((_CH, ODIM), jnp.float32),
            pltpu.SemaphoreType.DMA,
            pltpu.SemaphoreType.DMA,
            pltpu.SemaphoreType.DMA,
            pltpu.SemaphoreType.DMA,
        ],
    )
    def k(adduct_hbm, instr_hbm, apad_hbm, ipad_hbm, out_hbm,
          aidx_v, iidx_v, a0, a1, b0, b1, g0, g1, o0, o1):
        wid = lax.axis_index("s") * _NC + lax.axis_index("c")
        base = wid * _BPW
        row0 = wid * _NCHUNK
        pltpu.sync_copy(adduct_hbm.at[pl.ds(row0, _NCHUNK), :], aidx_v)
        pltpu.sync_copy(instr_hbm.at[pl.ds(row0, _NCHUNK), :], iidx_v)

        av = (a0, a1)
        bv = (b0, b1)
        gsem = (g0, g1)
        osem = (o0, o1)

        def fire(c):
            s = c % 2
            ca = pltpu.async_copy(apad_hbm.at[aidx_v.at[c]], av[s], gsem[s])
            cb = pltpu.async_copy(ipad_hbm.at[iidx_v.at[c]], bv[s], gsem[s])
            return ca, cb

        writes = [None, None]
        pend = fire(0)
        pending = [pend, None]
        for c in range(_NCHUNK):
            s = c % 2
            ns = (c + 1) % 2
            if c + 1 < _NCHUNK:
                if writes[ns] is not None:
                    writes[ns].wait()
                    writes[ns] = None
                pending[ns] = fire(c + 1)
            ca, cb = pending[s]
            ca.wait()
            cb.wait()

            def mergerow(r, _, _s=s):
                for k16 in range(DIM // 16):
                    sl = pl.ds(DIM + k16 * 16, 16)
                    av[_s][r, sl] = bv[_s][r, sl]
                return ()

            lax.fori_loop(0, _CH, mergerow, ())
            writes[s] = pltpu.async_copy(
                av[s], out_hbm.at[pl.ds(base + c * _CH, _CH), :], osem[s])
        for w in writes:
            if w is not None:
                w.wait()

    return k


_sc_kernel = _build()


def kernel(adduct, instrument_type, adduct_table, instrument_type_table):
    apad = jnp.pad(adduct_table, ((0, 0), (0, DIM)))
    ipad = jnp.pad(instrument_type_table, ((0, 0), (DIM, 0)))
    adduct2 = adduct.reshape(_NW * _NCHUNK, _CH)
    instr2 = instrument_type.reshape(_NW * _NCHUNK, _CH)
    return _sc_kernel(adduct2, instr2, apad, ipad)
